# baseline (speedup 1.0000x reference)
import functools

import jax
import jax.numpy as jnp
from jax import lax
from jax.experimental import pallas as pl
from jax.experimental.pallas import tpu as pltpu
from jax.experimental.pallas import tpu_sc as plsc

N_NODES = 10000
N_EDGES = 320000
D_FEAT = 128
LANES = 16
KS = D_FEAT // LANES

NUM_CORES = 2
NUM_SUBCORES = 16
NUM_WORKERS = NUM_CORES * NUM_SUBCORES
E_PER_W = N_EDGES // NUM_WORKERS
CHUNK = 80
NCHUNKS = E_PER_W // CHUNK
GROUPS = CHUNK // LANES


def _sc_edge_scorer(xw_hbm, x_hbm, src_hbm, dst_hbm, b_hbm, out_hbm,
                    si_v, di_v, sr_v, dr_v, o_v, b_v, sem0, sem1):
    wid = lax.axis_index("s") * NUM_CORES + lax.axis_index("c")
    base = wid * E_PER_W

    pltpu.sync_copy(src_hbm.at[pl.ds(base, E_PER_W)], si_v)
    pltpu.sync_copy(dst_hbm.at[pl.ds(base, E_PER_W)], di_v)
    pltpu.sync_copy(b_hbm, b_v)
    bvec = b_v[...]
    lane = lax.iota(jnp.int32, LANES)

    def issue(ci, buf, sem):
        eoff = ci * CHUNK
        boff = buf * CHUNK
        pltpu.async_copy(xw_hbm.at[si_v.at[pl.ds(eoff, CHUNK)]],
                         sr_v.at[pl.ds(boff, CHUNK)], sem)
        pltpu.async_copy(x_hbm.at[di_v.at[pl.ds(eoff, CHUNK)]],
                         dr_v.at[pl.ds(boff, CHUNK)], sem)

    def drain(ci, buf, sem):
        eoff = ci * CHUNK
        boff = buf * CHUNK
        pltpu.make_async_copy(xw_hbm.at[si_v.at[pl.ds(eoff, CHUNK)]],
                              sr_v.at[pl.ds(boff, CHUNK)], sem).wait()
        pltpu.make_async_copy(x_hbm.at[di_v.at[pl.ds(eoff, CHUNK)]],
                              dr_v.at[pl.ds(boff, CHUNK)], sem).wait()

    issue(0, 0, sem0)

    def chunk_body(ci, carry):
        cur = lax.rem(ci, 2)

        @pl.when(ci + 1 < NCHUNKS)
        def _():
            @pl.when(cur == 0)
            def _():
                issue(ci + 1, 1, sem1)

            @pl.when(cur == 1)
            def _():
                issue(ci + 1, 0, sem0)

        @pl.when(cur == 0)
        def _():
            drain(ci, 0, sem0)

        @pl.when(cur == 1)
        def _():
            drain(ci, 1, sem1)

        coff = cur * CHUNK
        ooff = ci * CHUNK

        @plsc.parallel_loop(0, GROUPS, 1, unroll=2)
        def group_body(g):
            e0 = coff + g * LANES
            parts = []
            for j in range(LANES):
                acc_a = jnp.zeros((LANES,), jnp.float32)
                acc_b = jnp.zeros((LANES,), jnp.float32)
                for k in range(KS // 2):
                    s = sr_v[e0 + j, pl.ds(k * LANES, LANES)]
                    t = dr_v[e0 + j, pl.ds(k * LANES, LANES)]
                    acc_a = acc_a + s * t
                for k in range(KS // 2, KS):
                    s = sr_v[e0 + j, pl.ds(k * LANES, LANES)]
                    t = dr_v[e0 + j, pl.ds(k * LANES, LANES)]
                    acc_b = acc_b + s * t
                c = plsc.cumsum(acc_a + acc_b)
                parts.append(jnp.where(lane == j, c[LANES - 1], 0.0))
            while len(parts) > 1:
                parts = [a + b for a, b in zip(parts[::2], parts[1::2])]
            z = parts[0] + bvec
            o_v[pl.ds(ooff + g * LANES, LANES)] = 1.0 / (1.0 + jnp.exp(-z))

        return carry

    lax.fori_loop(0, NCHUNKS, chunk_body, 0)
    pltpu.sync_copy(o_v, out_hbm.at[pl.ds(base, E_PER_W)])


def _tc_scale(x_ref, w_ref, o_ref):
    o_ref[...] = x_ref[...] * w_ref[...]


def kernel(x, edge_index, W, b):
    src = edge_index[0]
    dst = edge_index[1]
    b_vec = jnp.broadcast_to(b, (LANES,)).astype(jnp.float32)

    xw = pl.pallas_call(
        _tc_scale,
        out_shape=jax.ShapeDtypeStruct((N_NODES, D_FEAT), jnp.float32),
    )(x, W)

    mesh = plsc.VectorSubcoreMesh(core_axis_name="c", subcore_axis_name="s")
    run = functools.partial(
        pl.kernel,
        out_type=jax.ShapeDtypeStruct((N_EDGES,), jnp.float32),
        mesh=mesh,
        compiler_params=pltpu.CompilerParams(needs_layout_passes=False),
        scratch_types=[
            pltpu.VMEM((E_PER_W,), jnp.int32),
            pltpu.VMEM((E_PER_W,), jnp.int32),
            pltpu.VMEM((2 * CHUNK, D_FEAT), jnp.float32),
            pltpu.VMEM((2 * CHUNK, D_FEAT), jnp.float32),
            pltpu.VMEM((E_PER_W,), jnp.float32),
            pltpu.VMEM((LANES,), jnp.float32),
            pltpu.SemaphoreType.DMA,
            pltpu.SemaphoreType.DMA,
        ],
    )(_sc_edge_scorer)
    score = run(xw, x, src, dst, b_vec)
    return score.reshape(N_EDGES, 1)


# speedup vs baseline: 1.0556x; 1.0556x over previous
import functools

import jax
import jax.numpy as jnp
from jax import lax
from jax.experimental import pallas as pl
from jax.experimental.pallas import tpu as pltpu
from jax.experimental.pallas import tpu_sc as plsc

N_NODES = 10000
N_EDGES = 320000
D_FEAT = 128
LANES = 16
KS = D_FEAT // LANES

NUM_CORES = 2
NUM_SUBCORES = 16
NUM_WORKERS = NUM_CORES * NUM_SUBCORES
E_PER_W = N_EDGES // NUM_WORKERS
CHUNK = 80
NCHUNKS = E_PER_W // CHUNK
GROUPS = CHUNK // LANES


def _sc_edge_scorer(x_hbm, src_hbm, dst_hbm, w_hbm, b_hbm, out_hbm,
                    si_v, di_v, sr_v, dr_v, o_v, w_v, b_v, sem0, sem1):
    wid = lax.axis_index("s") * NUM_CORES + lax.axis_index("c")
    base = wid * E_PER_W

    pltpu.sync_copy(src_hbm.at[pl.ds(base, E_PER_W)], si_v)
    pltpu.sync_copy(dst_hbm.at[pl.ds(base, E_PER_W)], di_v)
    pltpu.sync_copy(w_hbm, w_v)
    pltpu.sync_copy(b_hbm, b_v)
    bvec = b_v[...]
    ws = [w_v[pl.ds(k * LANES, LANES)] for k in range(KS)]
    lane = lax.iota(jnp.int32, LANES)
    onehots = [(lane == j).astype(jnp.float32) for j in range(LANES)]

    def issue(ci, buf, sem):
        eoff = ci * CHUNK
        boff = buf * CHUNK
        pltpu.async_copy(x_hbm.at[si_v.at[pl.ds(eoff, CHUNK)]],
                         sr_v.at[pl.ds(boff, CHUNK)], sem)
        pltpu.async_copy(x_hbm.at[di_v.at[pl.ds(eoff, CHUNK)]],
                         dr_v.at[pl.ds(boff, CHUNK)], sem)

    def drain(ci, buf, sem):
        eoff = ci * CHUNK
        boff = buf * CHUNK
        pltpu.make_async_copy(x_hbm.at[si_v.at[pl.ds(eoff, CHUNK)]],
                              sr_v.at[pl.ds(boff, CHUNK)], sem).wait()
        pltpu.make_async_copy(x_hbm.at[di_v.at[pl.ds(eoff, CHUNK)]],
                              dr_v.at[pl.ds(boff, CHUNK)], sem).wait()

    issue(0, 0, sem0)

    def chunk_body(ci, carry):
        cur = lax.rem(ci, 2)

        @pl.when(ci + 1 < NCHUNKS)
        def _():
            @pl.when(cur == 0)
            def _():
                issue(ci + 1, 1, sem1)

            @pl.when(cur == 1)
            def _():
                issue(ci + 1, 0, sem0)

        @pl.when(cur == 0)
        def _():
            drain(ci, 0, sem0)

        @pl.when(cur == 1)
        def _():
            drain(ci, 1, sem1)

        coff = cur * CHUNK
        ooff = ci * CHUNK

        @plsc.parallel_loop(0, GROUPS, 1, unroll=2)
        def group_body(g):
            e0 = coff + g * LANES
            parts = []
            for j in range(LANES):
                acc_a = jnp.zeros((LANES,), jnp.float32)
                acc_b = jnp.zeros((LANES,), jnp.float32)
                for k in range(KS // 2):
                    s = sr_v[e0 + j, pl.ds(k * LANES, LANES)]
                    t = dr_v[e0 + j, pl.ds(k * LANES, LANES)]
                    acc_a = acc_a + (s * t) * ws[k]
                for k in range(KS // 2, KS):
                    s = sr_v[e0 + j, pl.ds(k * LANES, LANES)]
                    t = dr_v[e0 + j, pl.ds(k * LANES, LANES)]
                    acc_b = acc_b + (s * t) * ws[k]
                c = plsc.cumsum(acc_a + acc_b)
                parts.append(c[LANES - 1] * onehots[j])
            while len(parts) > 1:
                parts = [a + b for a, b in zip(parts[::2], parts[1::2])]
            z = parts[0] + bvec
            o_v[pl.ds(ooff + g * LANES, LANES)] = 1.0 / (1.0 + jnp.exp(-z))

        return carry

    lax.fori_loop(0, NCHUNKS, chunk_body, 0)
    pltpu.sync_copy(o_v, out_hbm.at[pl.ds(base, E_PER_W)])


def kernel(x, edge_index, W, b):
    src = edge_index[0]
    dst = edge_index[1]
    w_vec = W[0]
    b_vec = jnp.broadcast_to(b, (LANES,)).astype(jnp.float32)

    mesh = plsc.VectorSubcoreMesh(core_axis_name="c", subcore_axis_name="s")
    run = functools.partial(
        pl.kernel,
        out_type=jax.ShapeDtypeStruct((N_EDGES,), jnp.float32),
        mesh=mesh,
        compiler_params=pltpu.CompilerParams(needs_layout_passes=False),
        scratch_types=[
            pltpu.VMEM((E_PER_W,), jnp.int32),
            pltpu.VMEM((E_PER_W,), jnp.int32),
            pltpu.VMEM((2 * CHUNK, D_FEAT), jnp.float32),
            pltpu.VMEM((2 * CHUNK, D_FEAT), jnp.float32),
            pltpu.VMEM((E_PER_W,), jnp.float32),
            pltpu.VMEM((D_FEAT,), jnp.float32),
            pltpu.VMEM((LANES,), jnp.float32),
            pltpu.SemaphoreType.DMA,
            pltpu.SemaphoreType.DMA,
        ],
    )(_sc_edge_scorer)
    score = run(x, src, dst, w_vec, b_vec)
    return score.reshape(N_EDGES, 1)
